# baseline (speedup 1.0000x reference)
import functools

import jax
import jax.numpy as jnp
from jax import lax
from jax.experimental import pallas as pl
from jax.experimental.pallas import tpu as pltpu
from jax.experimental.pallas import tpu_sc as plsc

N_TOK = 8192
DM = 1024
HID = 1024
NE = 8
CAP = 2560
EC = NE * CAP
PAD = 256
ECP = EC + PAD
DUMP = EC
T = 512
NB = N_TOK // T
RB = 256
NMB = ECP // RB
BPE = CAP // RB


def _route_body(x_ref, wg_ref, idx0_ref, idx1_ref, w0_ref, w1_ref, cnt_ref,
                xp_ref, base_ref):
    i = pl.program_id(0)

    @pl.when(i == 0)
    def _():
        base_ref[...] = jnp.zeros_like(base_ref)

    logits = lax.dot_general(
        x_ref[...], wg_ref[...], (((1,), (0,)), ((), ())),
        preferred_element_type=jnp.float32)
    m = jnp.max(logits, axis=1, keepdims=True)
    eg = jnp.exp(logits - m)
    gates = eg / jnp.sum(eg, axis=1, keepdims=True)

    li = lax.broadcasted_iota(jnp.int32, (T, NE), 1).astype(jnp.float32)
    m1 = jnp.max(gates, axis=1, keepdims=True)
    idx1 = jnp.min(jnp.where(gates == m1, li, float(NE)), axis=1, keepdims=True)
    oh1 = (li == idx1).astype(jnp.float32)
    g2 = jnp.where(oh1 > 0, -jnp.inf, gates)
    m2 = jnp.max(g2, axis=1, keepdims=True)
    idx2 = jnp.min(jnp.where(g2 == m2, li, float(NE)), axis=1, keepdims=True)
    oh2 = (li == idx2).astype(jnp.float32)

    denom = m1 + m2 + 1e-9
    s1 = m1 / denom
    s2 = m2 / denom

    combined = oh1 + oh2
    r = lax.broadcasted_iota(jnp.int32, (T, T), 0)
    c = lax.broadcasted_iota(jnp.int32, (T, T), 1)
    tri = (r > c).astype(jnp.float32)
    excl = lax.dot_general(tri, combined, (((1,), (0,)), ((), ())),
                           preferred_element_type=jnp.float32)
    posm = excl + base_ref[...]
    pos1 = jnp.sum(posm * oh1, axis=1, keepdims=True)
    pos2 = jnp.sum(posm * oh2, axis=1, keepdims=True)
    base_ref[...] = base_ref[...] + jnp.sum(combined, axis=0, keepdims=True)

    def finish(posk, ohk, sk, idx_ref, w_ref):
        keep = (posk < CAP).astype(jnp.float32)
        ek = jnp.sum(li * ohk, axis=1, keepdims=True)
        slot = ek * CAP + jnp.minimum(posk, CAP - 1)
        slot = jnp.where(keep > 0, slot, float(DUMP))
        idx_ref[...] = slot.astype(jnp.int32)
        w_ref[...] = sk * keep

    finish(pos1, oh1, s1, idx0_ref, w0_ref)
    finish(pos2, oh2, s2, idx1_ref, w1_ref)
    cnt_ref[...] = base_ref[...]
    rb = x_ref[...].astype(jnp.bfloat16).astype(jnp.float32)
    bits = lax.bitcast_convert_type(rb, jnp.int32)
    top = jnp.bitwise_and(bits, jnp.int32(-65536))
    xp_ref[...] = jnp.bitwise_or(
        top[:, DM // 2:],
        lax.shift_right_logical(top[:, : DM // 2], 16))


def _route(x, wg, interpret=False):
    out1 = jax.ShapeDtypeStruct((N_TOK, 1), jnp.int32)
    outw = jax.ShapeDtypeStruct((N_TOK, 1), jnp.float32)
    return pl.pallas_call(
        _route_body,
        grid=(NB,),
        in_specs=[
            pl.BlockSpec((T, DM), lambda i: (i, 0)),
            pl.BlockSpec((DM, NE), lambda i: (0, 0)),
        ],
        out_specs=[pl.BlockSpec((T, 1), lambda i: (i, 0))] * 4
        + [pl.BlockSpec((1, NE), lambda i: (0, 0)),
           pl.BlockSpec((T, DM // 2), lambda i: (i, 0))],
        out_shape=[out1, out1, outw, outw,
                   jax.ShapeDtypeStruct((1, NE), jnp.float32),
                   jax.ShapeDtypeStruct((N_TOK, DM // 2), jnp.int32)],
        scratch_shapes=[pltpu.VMEM((1, NE), jnp.float32)],
        interpret=interpret,
    )(x, wg)


def _mlp_body(flags_ref, buf_ref, w1_ref, b1_ref, w2_ref, b2_ref, ws_ref,
              out_ref):
    i = pl.program_id(0)
    act = flags_ref[i] > 0

    @pl.when(act)
    def _():
        v = buf_ref[...]
        lo = lax.bitcast_convert_type(lax.shift_left(v, 16), jnp.float32)
        hi = lax.bitcast_convert_type(
            jnp.bitwise_and(v, jnp.int32(-65536)), jnp.float32)
        h = lax.dot_general(lo, w1_ref[0, : DM // 2, :],
                            (((1,), (0,)), ((), ())),
                            preferred_element_type=jnp.float32)
        h = h + lax.dot_general(hi, w1_ref[0, DM // 2:, :],
                                (((1,), (0,)), ((), ())),
                                preferred_element_type=jnp.float32)
        h = jnp.maximum(h + b1_ref[0], 0.0)
        o = lax.dot_general(h, w2_ref[0], (((1,), (0,)), ((), ())),
                            preferred_element_type=jnp.float32)
        o = (o + b2_ref[0]) * ws_ref[:, 0:1]
        ob = lax.bitcast_convert_type(
            o.astype(jnp.bfloat16).astype(jnp.float32), jnp.int32)
        top = jnp.bitwise_and(ob, jnp.int32(-65536))
        out_ref[...] = jnp.bitwise_or(
            top[:, DM // 2:],
            lax.shift_right_logical(top[:, : DM // 2], 16))

    @pl.when(jnp.logical_not(act))
    def _():
        out_ref[...] = jnp.zeros_like(out_ref)


def _mlp(buf, w1, b1, w2, b2, wslot, flags, interpret=False):
    def eix(i):
        return jnp.minimum(i // BPE, NE - 1)

    grid_spec = pltpu.PrefetchScalarGridSpec(
        num_scalar_prefetch=1,
        grid=(NMB,),
        in_specs=[
            pl.BlockSpec((RB, DM // 2), lambda i, f: (i, 0)),
            pl.BlockSpec((1, DM, HID), lambda i, f: (eix(i), 0, 0)),
            pl.BlockSpec((1, 1, HID), lambda i, f: (eix(i), 0, 0)),
            pl.BlockSpec((1, HID, DM), lambda i, f: (eix(i), 0, 0)),
            pl.BlockSpec((1, 1, DM), lambda i, f: (eix(i), 0, 0)),
            pl.BlockSpec((RB, 128), lambda i, f: (i, 0)),
        ],
        out_specs=pl.BlockSpec((RB, DM // 2), lambda i, f: (i, 0)),
    )
    return pl.pallas_call(
        _mlp_body,
        grid_spec=grid_spec,
        out_shape=jax.ShapeDtypeStruct((ECP, DM // 2), jnp.int32),
        interpret=interpret,
    )(flags, buf, w1, b1.reshape(NE, 1, HID), w2, b2.reshape(NE, 1, DM),
      wslot)


NW = 32
TPW = N_TOK // NW
SUBD = 32
NSUBD = TPW // SUBD
SUBC = 16
NSUBC = TPW // SUBC


def _dispatch(x, i0_3d, i1_3d, w0_3d, w1_3d):
    mesh = plsc.VectorSubcoreMesh(core_axis_name="c", subcore_axis_name="s")

    @functools.partial(
        pl.kernel, mesh=mesh,
        out_type=[jax.ShapeDtypeStruct((ECP, DM // 2), jnp.int32),
                  jax.ShapeDtypeStruct((ECP, 128), jnp.float32)],
        scratch_types=[
            pltpu.VMEM((SUBD, DM // 2), jnp.int32),
            pltpu.VMEM((SUBD, DM // 2), jnp.int32),
            pltpu.VMEM((SUBD,), jnp.int32),
            pltpu.VMEM((SUBD,), jnp.int32),
            pltpu.VMEM((SUBD,), jnp.int32),
            pltpu.VMEM((SUBD,), jnp.int32),
            pltpu.VMEM((SUBD,), jnp.float32),
            pltpu.VMEM((SUBD,), jnp.float32),
            pltpu.VMEM((SUBD,), jnp.float32),
            pltpu.VMEM((SUBD,), jnp.float32),
            pltpu.VMEM((SUBD, 128), jnp.float32),
            pltpu.VMEM((SUBD, 128), jnp.float32),
            pltpu.VMEM((SUBD, 128), jnp.float32),
            pltpu.VMEM((SUBD, 128), jnp.float32),
            pltpu.SemaphoreType.DMA,
            pltpu.SemaphoreType.DMA,
            pltpu.SemaphoreType.DMA,
            pltpu.SemaphoreType.DMA,
        ],
    )
    def k(x_hbm, i0_hbm, i1_hbm, w0_hbm, w1_hbm, buf_hbm, ws_hbm,
          xvA, xvB, i0A, i0B, i1A, i1B, w0A, w0B, w1A, w1B,
          wvaA, wvaB, wvbA, wvbB, semLA, semLB, semSA, semSB):
        wid = lax.axis_index("s") * 2 + lax.axis_index("c")
        base = wid * TPW
        B = [
            dict(xv=xvA, i0=i0A, i1=i1A, w0=w0A, w1=w1A, wva=wvaA,
                 wvb=wvbA, semL=semLA, semS=semSA),
            dict(xv=xvB, i0=i0B, i1=i1B, w0=w0B, w1=w1B, wva=wvaB,
                 wvb=wvbB, semL=semLB, semS=semSB),
        ]

        def load(c, b):
            pltpu.sync_copy(i0_hbm.at[wid, c], b["i0"])
            pltpu.sync_copy(i1_hbm.at[wid, c], b["i1"])
            pltpu.sync_copy(w0_hbm.at[wid, c], b["w0"])
            pltpu.sync_copy(w1_hbm.at[wid, c], b["w1"])
            return pltpu.async_copy(
                x_hbm.at[pl.ds(base + c * SUBD, SUBD)], b["xv"], b["semL"])

        def build_wv(b):
            for g in range(SUBD // 16):
                w16a = b["w0"][pl.ds(g * 16, 16)]
                w16b = b["w1"][pl.ds(g * 16, 16)]
                for r in range(16):
                    row = g * 16 + r
                    b["wva"][row, pl.ds(0, 16)] = lax.broadcast_in_dim(
                        w16a[r], (16,), ())
                    b["wvb"][row, pl.ds(0, 16)] = lax.broadcast_in_dim(
                        w16b[r], (16,), ())

        hL = load(0, B[0])
        scat = {0: None, 1: None}
        for c in range(NSUBD):
            b = B[c % 2]
            nb = B[(c + 1) % 2]
            if c + 1 < NSUBD:
                if scat[(c + 1) % 2] is not None:
                    for h in scat[(c + 1) % 2]:
                        h.wait()
                    scat[(c + 1) % 2] = None
                hL_next = load(c + 1, nb)
            build_wv(b)
            hL.wait()
            scat[c % 2] = [
                pltpu.async_copy(b["xv"], buf_hbm.at[b["i0"]], b["semS"]),
                pltpu.async_copy(b["xv"], buf_hbm.at[b["i1"]], b["semS"]),
                pltpu.async_copy(b["wva"], ws_hbm.at[b["i0"]], b["semS"]),
                pltpu.async_copy(b["wvb"], ws_hbm.at[b["i1"]], b["semS"]),
            ]
            if c + 1 < NSUBD:
                hL = hL_next
        for p in (0, 1):
            if scat[p] is not None:
                for h in scat[p]:
                    h.wait()

    return k(x, i0_3d, i1_3d, w0_3d, w1_3d)


def _combine(out, i0_3d, i1_3d):
    mesh = plsc.VectorSubcoreMesh(core_axis_name="c", subcore_axis_name="s")

    @functools.partial(
        pl.kernel, mesh=mesh,
        out_type=jax.ShapeDtypeStruct((N_TOK, DM), jnp.float32),
        scratch_types=[
            pltpu.VMEM((SUBC, DM // 2), jnp.int32),
            pltpu.VMEM((SUBC, DM // 2), jnp.int32),
            pltpu.VMEM((SUBC, DM // 2), jnp.int32),
            pltpu.VMEM((SUBC, DM // 2), jnp.int32),
            pltpu.VMEM((SUBC, DM), jnp.float32),
            pltpu.VMEM((SUBC, DM), jnp.float32),
            pltpu.VMEM((SUBC,), jnp.int32),
            pltpu.VMEM((SUBC,), jnp.int32),
            pltpu.VMEM((SUBC,), jnp.int32),
            pltpu.VMEM((SUBC,), jnp.int32),
            pltpu.SemaphoreType.DMA,
            pltpu.SemaphoreType.DMA,
        ],
    )
    def k(out_hbm, i0_hbm, i1_hbm, y_hbm,
          avA, avB, bvA, bvB, yvA, yvB, i0A, i0B, i1A, i1B, semA, semB):
        wid = lax.axis_index("s") * 2 + lax.axis_index("c")
        base = wid * TPW
        B = [
            dict(av=avA, bv=bvA, yv=yvA, i0=i0A, i1=i1A, sem=semA),
            dict(av=avB, bv=bvB, yv=yvB, i0=i0B, i1=i1B, sem=semB),
        ]

        def start(c, b):
            pltpu.sync_copy(i0_hbm.at[wid, c], b["i0"])
            pltpu.sync_copy(i1_hbm.at[wid, c], b["i1"])
            return (pltpu.async_copy(out_hbm.at[b["i0"]], b["av"], b["sem"]),
                    pltpu.async_copy(out_hbm.at[b["i1"]], b["bv"], b["sem"]))

        h = start(0, B[0])
        for c in range(NSUBC):
            b = B[c % 2]
            if c + 1 < NSUBC:
                h_next = start(c + 1, B[(c + 1) % 2])
            h[0].wait()
            h[1].wait()
            av, bv, yv = b["av"], b["bv"], b["yv"]

            def addrow(r, carry, av=av, bv=bv, yv=yv):
                for cc in range(DM // 32):
                    sl = pl.ds(cc * 16, 16)
                    va = av[r, sl]
                    vb = bv[r, sl]
                    lo = (lax.bitcast_convert_type(
                        lax.shift_left(va, 16), jnp.float32)
                        + lax.bitcast_convert_type(
                            lax.shift_left(vb, 16), jnp.float32))
                    hi = (lax.bitcast_convert_type(
                        jnp.bitwise_and(va, jnp.int32(-65536)), jnp.float32)
                        + lax.bitcast_convert_type(
                            jnp.bitwise_and(vb, jnp.int32(-65536)),
                            jnp.float32))
                    yv[r, sl] = lo
                    yv[r, pl.ds(DM // 2 + cc * 16, 16)] = hi
                return carry

            lax.fori_loop(0, SUBC, addrow, 0)
            pltpu.sync_copy(yv, y_hbm.at[pl.ds(base + c * SUBC, SUBC)])
            if c + 1 < NSUBC:
                h = h_next

    return k(out, i0_3d, i1_3d)


def kernel(x, Wg, W1, b1, W2, b2):
    idx0, idx1, w0, w1k, counts, xp = _route(x, Wg)
    i0d = idx0.reshape(NW, NSUBD, SUBD)
    i1d = idx1.reshape(NW, NSUBD, SUBD)
    w0d = w0.reshape(NW, NSUBD, SUBD)
    w1d = w1k.reshape(NW, NSUBD, SUBD)
    buf, wslot = _dispatch(xp, i0d, i1d, w0d, w1d)
    ids = jnp.arange(NMB, dtype=jnp.int32)
    e = jnp.minimum(ids // BPE, NE - 1)
    start_in_e = (ids - e * BPE) * RB
    active = (start_in_e.astype(jnp.float32) < counts[0][e]) & (ids < NMB - 1)
    flags = active.astype(jnp.int32)
    out = _mlp(buf, W1, b1, W2, b2, wslot, flags)
    i0c = idx0.reshape(NW, NSUBC, SUBC)
    i1c = idx1.reshape(NW, NSUBC, SUBC)
    return _combine(out, i0c, i1c)


# speedup vs baseline: 1.0662x; 1.0662x over previous
import functools

import jax
import jax.numpy as jnp
from jax import lax
from jax.experimental import pallas as pl
from jax.experimental.pallas import tpu as pltpu
from jax.experimental.pallas import tpu_sc as plsc

N_TOK = 8192
DM = 1024
HID = 1024
NE = 8
CAP = 2560
EC = NE * CAP
PAD = 256
ECP = EC + PAD
DUMP = EC
T = 512
NB = N_TOK // T
RB = 256
NMB = ECP // RB
BPE = CAP // RB


def _route_body(x_ref, wg_ref, idx0_ref, idx1_ref, w0_ref, w1_ref, cnt_ref,
                xp_ref, base_ref):
    i = pl.program_id(0)

    @pl.when(i == 0)
    def _():
        base_ref[...] = jnp.zeros_like(base_ref)

    logits = lax.dot_general(
        x_ref[...], wg_ref[...], (((1,), (0,)), ((), ())),
        preferred_element_type=jnp.float32)
    m = jnp.max(logits, axis=1, keepdims=True)
    eg = jnp.exp(logits - m)
    gates = eg / jnp.sum(eg, axis=1, keepdims=True)

    li = lax.broadcasted_iota(jnp.int32, (T, NE), 1).astype(jnp.float32)
    m1 = jnp.max(gates, axis=1, keepdims=True)
    idx1 = jnp.min(jnp.where(gates == m1, li, float(NE)), axis=1, keepdims=True)
    oh1 = (li == idx1).astype(jnp.float32)
    g2 = jnp.where(oh1 > 0, -jnp.inf, gates)
    m2 = jnp.max(g2, axis=1, keepdims=True)
    idx2 = jnp.min(jnp.where(g2 == m2, li, float(NE)), axis=1, keepdims=True)
    oh2 = (li == idx2).astype(jnp.float32)

    denom = m1 + m2 + 1e-9
    s1 = m1 / denom
    s2 = m2 / denom

    combined = oh1 + oh2
    r = lax.broadcasted_iota(jnp.int32, (T, T), 0)
    c = lax.broadcasted_iota(jnp.int32, (T, T), 1)
    tri = (r > c).astype(jnp.float32)
    excl = lax.dot_general(tri, combined, (((1,), (0,)), ((), ())),
                           preferred_element_type=jnp.float32)
    posm = excl + base_ref[...]
    pos1 = jnp.sum(posm * oh1, axis=1, keepdims=True)
    pos2 = jnp.sum(posm * oh2, axis=1, keepdims=True)
    base_ref[...] = base_ref[...] + jnp.sum(combined, axis=0, keepdims=True)

    def finish(posk, ohk, sk, idx_ref, w_ref):
        keep = (posk < CAP).astype(jnp.float32)
        ek = jnp.sum(li * ohk, axis=1, keepdims=True)
        slot = ek * CAP + jnp.minimum(posk, CAP - 1)
        slot = jnp.where(keep > 0, slot, float(DUMP))
        idx_ref[...] = slot.astype(jnp.int32)
        w_ref[...] = sk * keep

    finish(pos1, oh1, s1, idx0_ref, w0_ref)
    finish(pos2, oh2, s2, idx1_ref, w1_ref)
    cnt_ref[...] = base_ref[...]
    rb = x_ref[...].astype(jnp.bfloat16).astype(jnp.float32)
    bits = lax.bitcast_convert_type(rb, jnp.int32)
    top = jnp.bitwise_and(bits, jnp.int32(-65536))
    xp_ref[...] = jnp.bitwise_or(
        top[:, DM // 2:],
        lax.shift_right_logical(top[:, : DM // 2], 16))


def _route(x, wg, interpret=False):
    out1 = jax.ShapeDtypeStruct((N_TOK, 1), jnp.int32)
    outw = jax.ShapeDtypeStruct((N_TOK, 1), jnp.float32)
    return pl.pallas_call(
        _route_body,
        grid=(NB,),
        in_specs=[
            pl.BlockSpec((T, DM), lambda i: (i, 0)),
            pl.BlockSpec((DM, NE), lambda i: (0, 0)),
        ],
        out_specs=[pl.BlockSpec((T, 1), lambda i: (i, 0))] * 4
        + [pl.BlockSpec((1, NE), lambda i: (0, 0)),
           pl.BlockSpec((T, DM // 2), lambda i: (i, 0))],
        out_shape=[out1, out1, outw, outw,
                   jax.ShapeDtypeStruct((1, NE), jnp.float32),
                   jax.ShapeDtypeStruct((N_TOK, DM // 2), jnp.int32)],
        scratch_shapes=[pltpu.VMEM((1, NE), jnp.float32)],
        interpret=interpret,
    )(x, wg)


def _mlp_body(flags_ref, buf_ref, w1_ref, b1_ref, w2_ref, b2_ref, ws_ref,
              out_ref):
    i = pl.program_id(0)
    act = flags_ref[i] > 0

    @pl.when(act)
    def _():
        v = buf_ref[...]
        lo = lax.bitcast_convert_type(lax.shift_left(v, 16), jnp.float32)
        hi = lax.bitcast_convert_type(
            jnp.bitwise_and(v, jnp.int32(-65536)), jnp.float32)
        h = lax.dot_general(lo, w1_ref[0, : DM // 2, :],
                            (((1,), (0,)), ((), ())),
                            preferred_element_type=jnp.float32)
        h = h + lax.dot_general(hi, w1_ref[0, DM // 2:, :],
                                (((1,), (0,)), ((), ())),
                                preferred_element_type=jnp.float32)
        h = jnp.maximum(h + b1_ref[0], 0.0)
        o = lax.dot_general(h, w2_ref[0], (((1,), (0,)), ((), ())),
                            preferred_element_type=jnp.float32)
        o = o + b2_ref[0]
        out_ref[...] = o * ws_ref[:, 0:1]

    @pl.when(jnp.logical_not(act))
    def _():
        out_ref[...] = jnp.zeros_like(out_ref)


def _mlp(buf, w1, b1, w2, b2, wslot, flags, interpret=False):
    def eix(i):
        return jnp.minimum(i // BPE, NE - 1)

    grid_spec = pltpu.PrefetchScalarGridSpec(
        num_scalar_prefetch=1,
        grid=(NMB,),
        in_specs=[
            pl.BlockSpec((RB, DM // 2), lambda i, f: (i, 0)),
            pl.BlockSpec((1, DM, HID), lambda i, f: (eix(i), 0, 0)),
            pl.BlockSpec((1, 1, HID), lambda i, f: (eix(i), 0, 0)),
            pl.BlockSpec((1, HID, DM), lambda i, f: (eix(i), 0, 0)),
            pl.BlockSpec((1, 1, DM), lambda i, f: (eix(i), 0, 0)),
            pl.BlockSpec((RB, 128), lambda i, f: (i, 0)),
        ],
        out_specs=pl.BlockSpec((RB, DM), lambda i, f: (i, 0)),
    )
    return pl.pallas_call(
        _mlp_body,
        grid_spec=grid_spec,
        out_shape=jax.ShapeDtypeStruct((ECP, DM), jnp.float32),
        interpret=interpret,
    )(flags, buf, w1, b1.reshape(NE, 1, HID), w2, b2.reshape(NE, 1, DM),
      wslot)


NW = 32
TPW = N_TOK // NW
SUBD = 32
NSUBD = TPW // SUBD
SUBC = 16
NSUBC = TPW // SUBC


def _dispatch(x, i0_3d, i1_3d, w0_3d, w1_3d):
    mesh = plsc.VectorSubcoreMesh(core_axis_name="c", subcore_axis_name="s")

    @functools.partial(
        pl.kernel, mesh=mesh,
        out_type=[jax.ShapeDtypeStruct((ECP, DM // 2), jnp.int32),
                  jax.ShapeDtypeStruct((ECP, 128), jnp.float32)],
        scratch_types=[
            pltpu.VMEM((SUBD, DM // 2), jnp.int32),
            pltpu.VMEM((SUBD, DM // 2), jnp.int32),
            pltpu.VMEM((SUBD,), jnp.int32),
            pltpu.VMEM((SUBD,), jnp.int32),
            pltpu.VMEM((SUBD,), jnp.int32),
            pltpu.VMEM((SUBD,), jnp.int32),
            pltpu.VMEM((SUBD,), jnp.float32),
            pltpu.VMEM((SUBD,), jnp.float32),
            pltpu.VMEM((SUBD,), jnp.float32),
            pltpu.VMEM((SUBD,), jnp.float32),
            pltpu.VMEM((SUBD, 128), jnp.float32),
            pltpu.VMEM((SUBD, 128), jnp.float32),
            pltpu.VMEM((SUBD, 128), jnp.float32),
            pltpu.VMEM((SUBD, 128), jnp.float32),
            pltpu.SemaphoreType.DMA,
            pltpu.SemaphoreType.DMA,
            pltpu.SemaphoreType.DMA,
            pltpu.SemaphoreType.DMA,
        ],
    )
    def k(x_hbm, i0_hbm, i1_hbm, w0_hbm, w1_hbm, buf_hbm, ws_hbm,
          xvA, xvB, i0A, i0B, i1A, i1B, w0A, w0B, w1A, w1B,
          wvaA, wvaB, wvbA, wvbB, semLA, semLB, semSA, semSB):
        wid = lax.axis_index("s") * 2 + lax.axis_index("c")
        base = wid * TPW
        B = [
            dict(xv=xvA, i0=i0A, i1=i1A, w0=w0A, w1=w1A, wva=wvaA,
                 wvb=wvbA, semL=semLA, semS=semSA),
            dict(xv=xvB, i0=i0B, i1=i1B, w0=w0B, w1=w1B, wva=wvaB,
                 wvb=wvbB, semL=semLB, semS=semSB),
        ]

        def load(c, b):
            pltpu.sync_copy(i0_hbm.at[wid, c], b["i0"])
            pltpu.sync_copy(i1_hbm.at[wid, c], b["i1"])
            pltpu.sync_copy(w0_hbm.at[wid, c], b["w0"])
            pltpu.sync_copy(w1_hbm.at[wid, c], b["w1"])
            return pltpu.async_copy(
                x_hbm.at[pl.ds(base + c * SUBD, SUBD)], b["xv"], b["semL"])

        def build_wv(b):
            for g in range(SUBD // 16):
                w16a = b["w0"][pl.ds(g * 16, 16)]
                w16b = b["w1"][pl.ds(g * 16, 16)]
                for r in range(16):
                    row = g * 16 + r
                    b["wva"][row, pl.ds(0, 16)] = lax.broadcast_in_dim(
                        w16a[r], (16,), ())
                    b["wvb"][row, pl.ds(0, 16)] = lax.broadcast_in_dim(
                        w16b[r], (16,), ())

        hL = load(0, B[0])
        scat = {0: None, 1: None}
        for c in range(NSUBD):
            b = B[c % 2]
            nb = B[(c + 1) % 2]
            if c + 1 < NSUBD:
                if scat[(c + 1) % 2] is not None:
                    for h in scat[(c + 1) % 2]:
                        h.wait()
                    scat[(c + 1) % 2] = None
                hL_next = load(c + 1, nb)
            build_wv(b)
            hL.wait()
            scat[c % 2] = [
                pltpu.async_copy(b["xv"], buf_hbm.at[b["i0"]], b["semS"]),
                pltpu.async_copy(b["xv"], buf_hbm.at[b["i1"]], b["semS"]),
                pltpu.async_copy(b["wva"], ws_hbm.at[b["i0"]], b["semS"]),
                pltpu.async_copy(b["wvb"], ws_hbm.at[b["i1"]], b["semS"]),
            ]
            if c + 1 < NSUBD:
                hL = hL_next
        for p in (0, 1):
            if scat[p] is not None:
                for h in scat[p]:
                    h.wait()

    return k(x, i0_3d, i1_3d, w0_3d, w1_3d)


def _combine(out, i0_3d, i1_3d):
    mesh = plsc.VectorSubcoreMesh(core_axis_name="c", subcore_axis_name="s")

    @functools.partial(
        pl.kernel, mesh=mesh,
        out_type=jax.ShapeDtypeStruct((N_TOK, DM), jnp.float32),
        scratch_types=[
            pltpu.VMEM((SUBC, DM), jnp.float32),
            pltpu.VMEM((SUBC, DM), jnp.float32),
            pltpu.VMEM((SUBC, DM), jnp.float32),
            pltpu.VMEM((SUBC, DM), jnp.float32),
            pltpu.VMEM((SUBC,), jnp.int32),
            pltpu.VMEM((SUBC,), jnp.int32),
            pltpu.VMEM((SUBC,), jnp.int32),
            pltpu.VMEM((SUBC,), jnp.int32),
            pltpu.SemaphoreType.DMA,
            pltpu.SemaphoreType.DMA,
        ],
    )
    def k(out_hbm, i0_hbm, i1_hbm, y_hbm,
          avA, avB, bvA, bvB, i0A, i0B, i1A, i1B, semA, semB):
        wid = lax.axis_index("s") * 2 + lax.axis_index("c")
        base = wid * TPW
        B = [
            dict(av=avA, bv=bvA, i0=i0A, i1=i1A, sem=semA),
            dict(av=avB, bv=bvB, i0=i0B, i1=i1B, sem=semB),
        ]

        def start(c, b):
            pltpu.sync_copy(i0_hbm.at[wid, c], b["i0"])
            pltpu.sync_copy(i1_hbm.at[wid, c], b["i1"])
            return (pltpu.async_copy(out_hbm.at[b["i0"]], b["av"], b["sem"]),
                    pltpu.async_copy(out_hbm.at[b["i1"]], b["bv"], b["sem"]))

        h = start(0, B[0])
        for c in range(NSUBC):
            b = B[c % 2]
            if c + 1 < NSUBC:
                h_next = start(c + 1, B[(c + 1) % 2])
            h[0].wait()
            h[1].wait()
            av, bv = b["av"], b["bv"]

            def addrow(r, carry, av=av, bv=bv):
                for cc in range(DM // 16):
                    sl = pl.ds(cc * 16, 16)
                    av[r, sl] = av[r, sl] + bv[r, sl]
                return carry

            lax.fori_loop(0, SUBC, addrow, 0)
            pltpu.sync_copy(av, y_hbm.at[pl.ds(base + c * SUBC, SUBC)])
            if c + 1 < NSUBC:
                h = h_next

    return k(out, i0_3d, i1_3d)


def kernel(x, Wg, W1, b1, W2, b2):
    idx0, idx1, w0, w1k, counts, xp = _route(x, Wg)
    i0d = idx0.reshape(NW, NSUBD, SUBD)
    i1d = idx1.reshape(NW, NSUBD, SUBD)
    w0d = w0.reshape(NW, NSUBD, SUBD)
    w1d = w1k.reshape(NW, NSUBD, SUBD)
    buf, wslot = _dispatch(xp, i0d, i1d, w0d, w1d)
    ids = jnp.arange(NMB, dtype=jnp.int32)
    e = jnp.minimum(ids // BPE, NE - 1)
    start_in_e = (ids - e * BPE) * RB
    active = (start_in_e.astype(jnp.float32) < counts[0][e]) & (ids < NMB - 1)
    flags = active.astype(jnp.int32)
    out = _mlp(buf, W1, b1, W2, b2, wslot, flags)
    i0c = idx0.reshape(NW, NSUBC, SUBC)
    i1c = idx1.reshape(NW, NSUBC, SUBC)
    return _combine(out, i0c, i1c)
